# baseline (speedup 1.0000x reference)
import functools

import jax
import jax.numpy as jnp
from jax import lax
from jax.experimental import pallas as pl
from jax.experimental.pallas import tpu as pltpu
from jax.experimental.pallas import tpu_sc as plsc

N = 10000
E = 320000
D_IN = 128
D_H = 256
D_OUT = 128

NW = 32
CHUNK = 128
EP = 327680
CPT = EP // (NW * CHUNK)
PADN = N
NP = 10240
RPS = NP // 16

_mesh = plsc.VectorSubcoreMesh(core_axis_name="c", subcore_axis_name="s")


@functools.partial(
    pl.kernel,
    out_type=jax.ShapeDtypeStruct((2, NP), jnp.float32),
    mesh=_mesh,
    scratch_types=[
        pltpu.VMEM((CPT, CHUNK), jnp.int32),
        pltpu.VMEM((CHUNK,), jnp.float32),
        pltpu.VMEM_SHARED((NP,), jnp.float32),
    ],
    compiler_params=pltpu.CompilerParams(needs_layout_passes=False),
)
def _sc_degree(dst_hbm, zt_hbm, deg_out, dstv, ones_v, deg_sh):
    cid = lax.axis_index("c")
    sid = lax.axis_index("s")
    wid = sid * 2 + cid
    for k in range(CHUNK // 16):
        ones_v[pl.ds(k * 16, 16)] = jnp.ones((16,), jnp.float32)
    pltpu.sync_copy(zt_hbm.at[pl.ds(sid * RPS, RPS)],
                    deg_sh.at[pl.ds(sid * RPS, RPS)])
    pltpu.sync_copy(dst_hbm.at[pl.ds(wid * CPT, CPT)], dstv)
    plsc.subcore_barrier()

    @pl.loop(0, CPT)
    def _(j):
        pltpu.sync_copy(ones_v, deg_sh.at[dstv.at[j]], add=True)

    plsc.subcore_barrier()
    pltpu.sync_copy(deg_sh.at[pl.ds(sid * RPS, RPS)],
                    deg_out.at[cid].at[pl.ds(sid * RPS, RPS)])


@functools.partial(
    pl.kernel,
    out_type=(
        jax.ShapeDtypeStruct((2, NP, D_IN), jnp.float32),
        jax.ShapeDtypeStruct((2, NP), jnp.float32),
    ),
    mesh=_mesh,
    scratch_types=[
        pltpu.VMEM((CHUNK,), jnp.int32),
        pltpu.VMEM((CHUNK,), jnp.int32),
        pltpu.VMEM((CHUNK,), jnp.int32),
        pltpu.VMEM((CHUNK,), jnp.int32),
        pltpu.VMEM((CHUNK, D_IN), jnp.float32),
        pltpu.VMEM((CHUNK, D_IN), jnp.float32),
        pltpu.VMEM((CHUNK,), jnp.float32),
        pltpu.VMEM((CHUNK,), jnp.float32),
        pltpu.VMEM((NP,), jnp.float32),
        pltpu.VMEM_SHARED((NP, D_IN), jnp.float32),
        pltpu.VMEM_SHARED((NP,), jnp.float32),
        pltpu.SemaphoreType.DMA,
        pltpu.SemaphoreType.DMA,
        pltpu.SemaphoreType.DMA,
        pltpu.SemaphoreType.DMA,
        pltpu.SemaphoreType.DMA,
        pltpu.SemaphoreType.DMA,
        pltpu.SemaphoreType.DMA,
        pltpu.SemaphoreType.DMA,
    ],
    compiler_params=pltpu.CompilerParams(needs_layout_passes=False),
)
def _sc_edge_pass(src_hbm, dst_hbm, y_hbm, dis_hbm, zs_hbm, zt_hbm,
                  s_out, t_out, idxs_a, idxs_b, idxd_a, idxd_b,
                  rows_a, rows_b, dvals_a, dvals_b, dis_loc, s_sh, t_sh,
                  sg_a, sg_b, ss_a, ss_b, si_a, si_b, st_a, st_b):
    cid = lax.axis_index("c")
    sid = lax.axis_index("s")
    wid = sid * 2 + cid
    base = wid * CPT * CHUNK
    pltpu.sync_copy(zs_hbm.at[pl.ds(sid * RPS, RPS)],
                    s_sh.at[pl.ds(sid * RPS, RPS)])
    pltpu.sync_copy(zt_hbm.at[pl.ds(sid * RPS, RPS)],
                    t_sh.at[pl.ds(sid * RPS, RPS)])
    pltpu.sync_copy(dis_hbm, dis_loc)
    plsc.subcore_barrier()

    idxs = (idxs_a, idxs_b)
    idxd = (idxd_a, idxd_b)
    rows = (rows_a, rows_b)
    dvals = (dvals_a, dvals_b)
    sg = (sg_a, sg_b)
    ss = (ss_a, ss_b)
    si = (si_a, si_b)
    st = (st_a, st_b)
    pltpu.sync_copy(src_hbm.at[pl.ds(base, CHUNK)], idxs_a)
    pltpu.sync_copy(dst_hbm.at[pl.ds(base, CHUNK)], idxd_a)
    pltpu.async_copy(y_hbm.at[idxs_a], rows_a, sg_a)

    @pl.loop(0, CPT, step=2)
    def _(j):
        for b in range(2):
            jj = j + b
            o = 1 - b

            @pl.when(jj > 0)
            def _():
                pltpu.make_async_copy(
                    rows[o], s_sh.at[idxd[o]], ss[o]).wait()
                pltpu.make_async_copy(
                    dvals[o], t_sh.at[idxs[o]], st[o]).wait()

            @pl.when(jj + 1 < CPT)
            def _():
                off = base + (jj + 1) * CHUNK
                pltpu.async_copy(src_hbm.at[pl.ds(off, CHUNK)],
                                 idxs[o], si[o])
                pltpu.async_copy(dst_hbm.at[pl.ds(off, CHUNK)],
                                 idxd[o], si[o])

            pltpu.make_async_copy(y_hbm.at[idxs[b]], rows[b],
                                  sg[b]).wait()
            pltpu.async_copy(rows[b], s_sh.at[idxd[b]], ss[b], add=True)

            @pl.when(jj + 1 < CPT)
            def _():
                pltpu.make_async_copy(src_hbm.at[pl.ds(0, CHUNK)],
                                      idxs[o], si[o]).wait()
                pltpu.make_async_copy(dst_hbm.at[pl.ds(0, CHUNK)],
                                      idxd[o], si[o]).wait()
                pltpu.async_copy(y_hbm.at[idxs[o]], rows[o], sg[o])

            for k in range(CHUNK // 16):
                idx = idxd[b][pl.ds(k * 16, 16)]
                dvals[b][pl.ds(k * 16, 16)] = plsc.load_gather(
                    dis_loc, [idx])
            pltpu.async_copy(dvals[b], t_sh.at[idxs[b]], st[b], add=True)

    pltpu.make_async_copy(
        rows[1], s_sh.at[idxd[1]], ss[1]).wait()
    pltpu.make_async_copy(
        dvals[1], t_sh.at[idxs[1]], st[1]).wait()

    plsc.subcore_barrier()
    pltpu.sync_copy(s_sh.at[pl.ds(sid * RPS, RPS)],
                    s_out.at[cid].at[pl.ds(sid * RPS, RPS)])
    pltpu.sync_copy(t_sh.at[pl.ds(sid * RPS, RPS)],
                    t_out.at[cid].at[pl.ds(sid * RPS, RPS)])


def _tc_prep_body(deg0_ref, deg1_ref, x_ref, dis_ref, y_ref):
    deg = deg0_ref[...] + deg1_ref[...] + 1.0
    dis = jnp.where(deg > 0.0, lax.rsqrt(deg), 0.0)
    dis_ref[...] = dis
    y_ref[...] = x_ref[...] * dis


_tc_prep = pl.pallas_call(
    _tc_prep_body,
    out_shape=(
        jax.ShapeDtypeStruct((NP, 1), jnp.float32),
        jax.ShapeDtypeStruct((NP, D_IN), jnp.float32),
    ),
)

_RB = 400
_NBLK = N // _RB


def _tc_main_body(s0_ref, s1_ref, y_ref, dis_ref, t0_ref, t1_ref,
                  W1_ref, b1_ref, W2_ref, b2_ref, out_ref, zacc_ref):
    i = pl.program_id(0)
    dis = dis_ref[...]
    agg = dis * (s0_ref[...] + s1_ref[...] + y_ref[...])
    h1 = jnp.dot(agg, W1_ref[...], preferred_element_type=jnp.float32)
    h1 = jnp.maximum(h1 + b1_ref[...], 0.0)
    w = dis * (t0_ref[...] + t1_ref[...] + dis)
    part = jnp.sum(h1 * w, axis=0, keepdims=True)

    @pl.when(i == 0)
    def _():
        zacc_ref[...] = part

    @pl.when(i > 0)
    def _():
        zacc_ref[...] = zacc_ref[...] + part

    @pl.when(i == _NBLK - 1)
    def _():
        z = jnp.dot(zacc_ref[...], W2_ref[...],
                    preferred_element_type=jnp.float32)
        out_ref[...] = z * (1.0 / N) + b2_ref[...]


_tc_main = pl.pallas_call(
    _tc_main_body,
    grid=(_NBLK,),
    in_specs=[
        pl.BlockSpec((_RB, D_IN), lambda i: (i, 0)),
        pl.BlockSpec((_RB, D_IN), lambda i: (i, 0)),
        pl.BlockSpec((_RB, D_IN), lambda i: (i, 0)),
        pl.BlockSpec((_RB, 1), lambda i: (i, 0)),
        pl.BlockSpec((_RB, 1), lambda i: (i, 0)),
        pl.BlockSpec((_RB, 1), lambda i: (i, 0)),
        pl.BlockSpec((D_IN, D_H), lambda i: (0, 0)),
        pl.BlockSpec((1, D_H), lambda i: (0, 0)),
        pl.BlockSpec((D_H, D_OUT), lambda i: (0, 0)),
        pl.BlockSpec((1, D_OUT), lambda i: (0, 0)),
    ],
    out_specs=pl.BlockSpec((1, D_OUT), lambda i: (0, 0)),
    out_shape=jax.ShapeDtypeStruct((1, D_OUT), jnp.float32),
    scratch_shapes=[pltpu.VMEM((1, D_H), jnp.float32)],
)


@jax.jit
def kernel(x, edge_index, W1, b1, W2, b2):
    pad = PADN + (jnp.arange(EP - E, dtype=jnp.int32) % (NP - N))
    src1d = jnp.concatenate([edge_index[0], pad])
    dst1d = jnp.concatenate([edge_index[1], pad])
    dst2d = dst1d.reshape(EP // CHUNK, CHUNK)
    x_ext = jnp.zeros((NP, D_IN), jnp.float32).at[:N].set(x)
    zt1 = jnp.zeros((NP,), jnp.float32)
    zs = jnp.zeros((NP, D_IN), jnp.float32)

    deg_parts = _sc_degree(dst2d, zt1)
    dis, y = _tc_prep(deg_parts[0].reshape(NP, 1),
                      deg_parts[1].reshape(NP, 1), x_ext)
    s_parts, t_parts = _sc_edge_pass(src1d, dst1d, y, dis.reshape(NP),
                                     zs, zt1)

    return _tc_main(s_parts[0], s_parts[1], y, dis,
                    t_parts[0].reshape(NP, 1), t_parts[1].reshape(NP, 1),
                    W1, b1.reshape(1, D_H), W2, b2.reshape(1, D_OUT))


# speedup vs baseline: 1.0731x; 1.0731x over previous
import functools

import jax
import jax.numpy as jnp
from jax import lax
from jax.experimental import pallas as pl
from jax.experimental.pallas import tpu as pltpu
from jax.experimental.pallas import tpu_sc as plsc

N = 10000
E = 320000
D_IN = 128
D_H = 256
D_OUT = 128

NW = 32
CHUNK = 128
EP = 327680
CPT = EP // (NW * CHUNK)
PADN = N
NP = 10240
RPS = NP // 16

_mesh = plsc.VectorSubcoreMesh(core_axis_name="c", subcore_axis_name="s")


@functools.partial(
    pl.kernel,
    out_type=jax.ShapeDtypeStruct((2, NP), jnp.float32),
    mesh=_mesh,
    scratch_types=[
        pltpu.VMEM((CPT, CHUNK), jnp.int32),
        pltpu.VMEM((CHUNK,), jnp.float32),
        pltpu.VMEM_SHARED((NP,), jnp.float32),
        pltpu.SemaphoreType.DMA,
    ],
    compiler_params=pltpu.CompilerParams(needs_layout_passes=False),
)
def _sc_degree(dst_hbm, zt_hbm, deg_out, dstv, ones_v, deg_sh, sd):
    cid = lax.axis_index("c")
    sid = lax.axis_index("s")
    wid = sid * 2 + cid
    for k in range(CHUNK // 16):
        ones_v[pl.ds(k * 16, 16)] = jnp.ones((16,), jnp.float32)
    pltpu.sync_copy(zt_hbm.at[pl.ds(sid * RPS, RPS)],
                    deg_sh.at[pl.ds(sid * RPS, RPS)])
    pltpu.sync_copy(dst_hbm.at[pl.ds(wid * CPT, CPT)], dstv)
    plsc.subcore_barrier()

    @pl.loop(0, CPT)
    def _(j):
        pltpu.async_copy(ones_v, deg_sh.at[dstv.at[j]], sd, add=True)

    @pl.loop(0, CPT)
    def _(j):
        pltpu.make_async_copy(ones_v, deg_sh.at[dstv.at[0]], sd).wait()

    plsc.subcore_barrier()
    pltpu.sync_copy(deg_sh.at[pl.ds(sid * RPS, RPS)],
                    deg_out.at[cid].at[pl.ds(sid * RPS, RPS)])


@functools.partial(
    pl.kernel,
    out_type=(
        jax.ShapeDtypeStruct((2, NP, D_IN), jnp.float32),
        jax.ShapeDtypeStruct((2, NP), jnp.float32),
    ),
    mesh=_mesh,
    scratch_types=[
        pltpu.VMEM((CHUNK,), jnp.int32),
        pltpu.VMEM((CHUNK,), jnp.int32),
        pltpu.VMEM((CHUNK,), jnp.int32),
        pltpu.VMEM((CHUNK,), jnp.int32),
        pltpu.VMEM((CHUNK, D_IN), jnp.float32),
        pltpu.VMEM((CHUNK, D_IN), jnp.float32),
        pltpu.VMEM((CHUNK,), jnp.float32),
        pltpu.VMEM((CHUNK,), jnp.float32),
        pltpu.VMEM((NP,), jnp.float32),
        pltpu.VMEM_SHARED((NP, D_IN), jnp.float32),
        pltpu.VMEM_SHARED((NP,), jnp.float32),
        pltpu.SemaphoreType.DMA,
        pltpu.SemaphoreType.DMA,
        pltpu.SemaphoreType.DMA,
        pltpu.SemaphoreType.DMA,
        pltpu.SemaphoreType.DMA,
        pltpu.SemaphoreType.DMA,
        pltpu.SemaphoreType.DMA,
        pltpu.SemaphoreType.DMA,
    ],
    compiler_params=pltpu.CompilerParams(needs_layout_passes=False),
)
def _sc_edge_pass(src_hbm, dst_hbm, y_hbm, dis_hbm, zs_hbm, zt_hbm,
                  s_out, t_out, idxs_a, idxs_b, idxd_a, idxd_b,
                  rows_a, rows_b, dvals_a, dvals_b, dis_loc, s_sh, t_sh,
                  sg_a, sg_b, ss_a, ss_b, si_a, si_b, st_a, st_b):
    cid = lax.axis_index("c")
    sid = lax.axis_index("s")
    wid = sid * 2 + cid
    base = wid * CPT * CHUNK
    pltpu.sync_copy(zs_hbm.at[pl.ds(sid * RPS, RPS)],
                    s_sh.at[pl.ds(sid * RPS, RPS)])
    pltpu.sync_copy(zt_hbm.at[pl.ds(sid * RPS, RPS)],
                    t_sh.at[pl.ds(sid * RPS, RPS)])
    pltpu.sync_copy(dis_hbm, dis_loc)
    plsc.subcore_barrier()

    idxs = (idxs_a, idxs_b)
    idxd = (idxd_a, idxd_b)
    rows = (rows_a, rows_b)
    dvals = (dvals_a, dvals_b)
    sg = (sg_a, sg_b)
    ss = (ss_a, ss_b)
    si = (si_a, si_b)
    st = (st_a, st_b)
    pltpu.sync_copy(src_hbm.at[pl.ds(base, CHUNK)], idxs_a)
    pltpu.sync_copy(dst_hbm.at[pl.ds(base, CHUNK)], idxd_a)
    pltpu.async_copy(y_hbm.at[idxs_a], rows_a, sg_a)

    @pl.loop(0, CPT, step=2)
    def _(j):
        for b in range(2):
            jj = j + b
            o = 1 - b

            @pl.when(jj > 0)
            def _():
                pltpu.make_async_copy(
                    rows[o], s_sh.at[idxd[o]], ss[o]).wait()
                pltpu.make_async_copy(
                    dvals[o], t_sh.at[idxs[o]], st[o]).wait()

            @pl.when(jj + 1 < CPT)
            def _():
                off = base + (jj + 1) * CHUNK
                pltpu.async_copy(src_hbm.at[pl.ds(off, CHUNK)],
                                 idxs[o], si[o])
                pltpu.async_copy(dst_hbm.at[pl.ds(off, CHUNK)],
                                 idxd[o], si[o])

            pltpu.make_async_copy(y_hbm.at[idxs[b]], rows[b],
                                  sg[b]).wait()
            pltpu.async_copy(rows[b], s_sh.at[idxd[b]], ss[b], add=True)

            @pl.when(jj + 1 < CPT)
            def _():
                pltpu.make_async_copy(src_hbm.at[pl.ds(0, CHUNK)],
                                      idxs[o], si[o]).wait()
                pltpu.make_async_copy(dst_hbm.at[pl.ds(0, CHUNK)],
                                      idxd[o], si[o]).wait()
                pltpu.async_copy(y_hbm.at[idxs[o]], rows[o], sg[o])

            for k in range(CHUNK // 16):
                idx = idxd[b][pl.ds(k * 16, 16)]
                dvals[b][pl.ds(k * 16, 16)] = plsc.load_gather(
                    dis_loc, [idx])
            pltpu.async_copy(dvals[b], t_sh.at[idxs[b]], st[b], add=True)

    pltpu.make_async_copy(
        rows[1], s_sh.at[idxd[1]], ss[1]).wait()
    pltpu.make_async_copy(
        dvals[1], t_sh.at[idxs[1]], st[1]).wait()

    plsc.subcore_barrier()
    pltpu.sync_copy(s_sh.at[pl.ds(sid * RPS, RPS)],
                    s_out.at[cid].at[pl.ds(sid * RPS, RPS)])
    pltpu.sync_copy(t_sh.at[pl.ds(sid * RPS, RPS)],
                    t_out.at[cid].at[pl.ds(sid * RPS, RPS)])


def _tc_prep_body(deg0_ref, deg1_ref, x_ref, dis_ref, y_ref):
    deg = deg0_ref[...] + deg1_ref[...] + 1.0
    dis = jnp.where(deg > 0.0, lax.rsqrt(deg), 0.0)
    dis_ref[...] = dis
    y_ref[...] = x_ref[...] * dis


_tc_prep = pl.pallas_call(
    _tc_prep_body,
    out_shape=(
        jax.ShapeDtypeStruct((NP, 1), jnp.float32),
        jax.ShapeDtypeStruct((NP, D_IN), jnp.float32),
    ),
)

_RB = 2000
_NBLK = N // _RB


def _tc_main_body(s0_ref, s1_ref, y_ref, dis_ref, t0_ref, t1_ref,
                  W1_ref, b1_ref, W2_ref, b2_ref, out_ref, zacc_ref):
    i = pl.program_id(0)
    dis = dis_ref[...]
    agg = dis * (s0_ref[...] + s1_ref[...] + y_ref[...])
    h1 = jnp.dot(agg, W1_ref[...], preferred_element_type=jnp.float32)
    h1 = jnp.maximum(h1 + b1_ref[...], 0.0)
    w = dis * (t0_ref[...] + t1_ref[...] + dis)
    part = jnp.sum(h1 * w, axis=0, keepdims=True)

    @pl.when(i == 0)
    def _():
        zacc_ref[...] = part

    @pl.when(i > 0)
    def _():
        zacc_ref[...] = zacc_ref[...] + part

    @pl.when(i == _NBLK - 1)
    def _():
        z = jnp.dot(zacc_ref[...], W2_ref[...],
                    preferred_element_type=jnp.float32)
        out_ref[...] = z * (1.0 / N) + b2_ref[...]


_tc_main = pl.pallas_call(
    _tc_main_body,
    grid=(_NBLK,),
    in_specs=[
        pl.BlockSpec((_RB, D_IN), lambda i: (i, 0)),
        pl.BlockSpec((_RB, D_IN), lambda i: (i, 0)),
        pl.BlockSpec((_RB, D_IN), lambda i: (i, 0)),
        pl.BlockSpec((_RB, 1), lambda i: (i, 0)),
        pl.BlockSpec((_RB, 1), lambda i: (i, 0)),
        pl.BlockSpec((_RB, 1), lambda i: (i, 0)),
        pl.BlockSpec((D_IN, D_H), lambda i: (0, 0)),
        pl.BlockSpec((1, D_H), lambda i: (0, 0)),
        pl.BlockSpec((D_H, D_OUT), lambda i: (0, 0)),
        pl.BlockSpec((1, D_OUT), lambda i: (0, 0)),
    ],
    out_specs=pl.BlockSpec((1, D_OUT), lambda i: (0, 0)),
    out_shape=jax.ShapeDtypeStruct((1, D_OUT), jnp.float32),
    scratch_shapes=[pltpu.VMEM((1, D_H), jnp.float32)],
)


@jax.jit
def kernel(x, edge_index, W1, b1, W2, b2):
    pad = PADN + (jnp.arange(EP - E, dtype=jnp.int32) % (NP - N))
    src1d = jnp.concatenate([edge_index[0], pad])
    dst1d = jnp.concatenate([edge_index[1], pad])
    dst2d = dst1d.reshape(EP // CHUNK, CHUNK)
    x_ext = jnp.zeros((NP, D_IN), jnp.float32).at[:N].set(x)
    zt1 = jnp.zeros((NP,), jnp.float32)
    zs = jnp.zeros((NP, D_IN), jnp.float32)

    deg_parts = _sc_degree(dst2d, zt1)
    dis, y = _tc_prep(deg_parts[0].reshape(NP, 1),
                      deg_parts[1].reshape(NP, 1), x_ext)
    s_parts, t_parts = _sc_edge_pass(src1d, dst1d, y, dis.reshape(NP),
                                     zs, zt1)

    return _tc_main(s_parts[0], s_parts[1], y, dis,
                    t_parts[0].reshape(NP, 1), t_parts[1].reshape(NP, 1),
                    W1, b1.reshape(1, D_H), W2, b2.reshape(1, D_OUT))
